# Initial kernel scaffold; baseline (speedup 1.0000x reference)
import jax
import jax.numpy as jnp
from jax.experimental import pallas as pl


def kernel(x_0, x_1, x_2, adj_0, adj_1, inc_1, inc_2, inv_rr_0, inv_rr_1, inv_rrm1_1, inv_rrm1_2, params):
    raise NotImplementedError("write your pallas kernel here")


# speedup vs baseline: 1.1210x; 1.1210x over previous
import functools

import jax
import jax.numpy as jnp
from jax import lax
from jax.experimental import pallas as pl
from jax.experimental.pallas import tpu as pltpu
from jax.experimental.pallas import tpu_sc as plsc

C = 128
N0, N1, N2 = 10000, 40000, 20000
NC, NS = 2, 16
NW = NC * NS
KG = 256
BIG = 1 << 30


def _ceil_to(x, m):
    return -(-x // m) * m


def _mm_body(x_ref, w_ref, b_ref, o_ref):
    o_ref[...] = jnp.dot(x_ref[...], w_ref[...],
                         preferred_element_type=jnp.float32) + b_ref[...]


def _mm(x, w, b, bn):
    n, k = x.shape
    m = w.shape[1]
    return pl.pallas_call(
        _mm_body,
        grid=(n // bn,),
        in_specs=[pl.BlockSpec((bn, k), lambda i: (i, 0)),
                  pl.BlockSpec((k, m), lambda i: (0, 0)),
                  pl.BlockSpec((1, m), lambda i: (0, 0))],
        out_specs=pl.BlockSpec((bn, m), lambda i: (i, 0)),
        out_shape=jax.ShapeDtypeStruct((n, m), jnp.float32),
        compiler_params=pltpu.CompilerParams(
            dimension_semantics=("parallel",)),
    )(x, w, b.reshape(1, m))


def _edge_body(kinv, ga_ref, gb_ref, invT_ref, w1c_ref, w2_ref, b2_ref,
               winfT_ref, binf_ref, o_ref):
    g = ga_ref[...] + gb_ref[...]
    invT = invT_ref[...]
    w1c = w1c_ref[...]
    for k in range(kinv):
        g += invT[k, :][:, None] * w1c[k, :][None, :]
    h = jax.nn.silu(g)
    h2 = jax.nn.silu(jnp.dot(h, w2_ref[...],
                             preferred_element_type=jnp.float32) + b2_ref[...])
    gate = jax.nn.sigmoid(
        jnp.sum(h2 * winfT_ref[...], axis=1, keepdims=True) + binf_ref[0, 0])
    o_ref[...] = h2 * gate


def _edge_stage(ga, gb, invT, p, kinv, be=2048):
    ep = ga.shape[0]
    w1c = jnp.zeros((8, C), jnp.float32).at[:kinv].set(p['W1'][2 * C:2 * C + kinv])
    body = functools.partial(_edge_body, kinv)
    return pl.pallas_call(
        body,
        grid=(ep // be,),
        in_specs=[pl.BlockSpec((be, C), lambda i: (i, 0)),
                  pl.BlockSpec((be, C), lambda i: (i, 0)),
                  pl.BlockSpec((8, be), lambda i: (0, i)),
                  pl.BlockSpec((8, C), lambda i: (0, 0)),
                  pl.BlockSpec((C, C), lambda i: (0, 0)),
                  pl.BlockSpec((1, C), lambda i: (0, 0)),
                  pl.BlockSpec((1, C), lambda i: (0, 0)),
                  pl.BlockSpec((1, 1), lambda i: (0, 0))],
        out_specs=pl.BlockSpec((be, C), lambda i: (i, 0)),
        out_shape=jax.ShapeDtypeStruct((ep, C), jnp.float32),
        compiler_params=pltpu.CompilerParams(
            dimension_semantics=("parallel",)),
    )(ga, gb, invT, w1c, p['W2'], p['b2'].reshape(1, C),
      p['Winf'].reshape(1, C), p['binf'].reshape(1, 1))


def _upd_body(p_ref, x_ref, u1_ref, ub1_ref, u2_ref, ub2_ref, o_ref):
    agg = p_ref[0] + p_ref[1]
    a = jax.nn.silu(agg)
    t = jax.nn.silu(jnp.dot(a, u1_ref[...],
                            preferred_element_type=jnp.float32) + ub1_ref[...])
    o_ref[...] = x_ref[...] + jnp.dot(
        t, u2_ref[...], preferred_element_type=jnp.float32) + ub2_ref[...]


def _update(parts, x, u, bn=1000):
    n = x.shape[0]
    return pl.pallas_call(
        _upd_body,
        grid=(n // bn,),
        in_specs=[pl.BlockSpec((2, bn, C), lambda i: (0, i, 0)),
                  pl.BlockSpec((bn, C), lambda i: (i, 0)),
                  pl.BlockSpec((C, C), lambda i: (0, 0)),
                  pl.BlockSpec((1, C), lambda i: (0, 0)),
                  pl.BlockSpec((C, C), lambda i: (0, 0)),
                  pl.BlockSpec((1, C), lambda i: (0, 0))],
        out_specs=pl.BlockSpec((bn, C), lambda i: (i, 0)),
        out_shape=jax.ShapeDtypeStruct((n, C), jnp.float32),
        compiler_params=pltpu.CompilerParams(
            dimension_semantics=("parallel",)),
    )(parts, x, u['U1'], u['ub1'].reshape(1, C),
      u['U2'], u['ub2'].reshape(1, C))


def _sc_gather(tables, idxs):
    nj = len(tables)
    epads = [int(i.shape[0]) for i in idxs]
    mesh = plsc.VectorSubcoreMesh(core_axis_name="c", subcore_axis_name="s")
    out_type = [jax.ShapeDtypeStruct((ep, C), jnp.float32) for ep in epads]

    @functools.partial(
        pl.kernel, mesh=mesh, out_type=out_type,
        scratch_types=[pltpu.VMEM((KG,), jnp.int32),
                       pltpu.VMEM((KG, C), jnp.float32),
                       pltpu.SemaphoreType.DMA])
    def gather_kernel(*refs):
        table_refs = refs[:nj]
        idx_refs = refs[nj:2 * nj]
        out_refs = refs[2 * nj:3 * nj]
        idx_v, rows_v, sem = refs[3 * nj:]
        wid = lax.axis_index("s") * NC + lax.axis_index("c")
        for j in range(nj):
            nblk = epads[j] // (NW * KG)

            @pl.loop(0, nblk)
            def _(i, j=j, nblk=nblk):
                base = (wid * nblk + i) * KG
                pltpu.sync_copy(idx_refs[j].at[pl.ds(base, KG)], idx_v)
                pltpu.async_copy(table_refs[j].at[idx_v], rows_v, sem).wait()
                pltpu.sync_copy(rows_v, out_refs[j].at[pl.ds(base, KG)])

    return gather_kernel(*tables, *idxs)


def _sc_scatter(rank_jobs, npads, chs, accs):
    nr = len(rank_jobs)
    accmax = max(accs)
    mesh = plsc.VectorSubcoreMesh(core_axis_name="c", subcore_axis_name="s")
    out_type = [jax.ShapeDtypeStruct((NC, npads[r], C), jnp.float32)
                for r in range(nr)]
    flat_in = []
    for jobs in rank_jobs:
        for m, rv in jobs:
            flat_in.extend((m, rv))
    nin = len(flat_in)

    @functools.partial(
        pl.kernel, mesh=mesh, out_type=out_type,
        scratch_types=[pltpu.VMEM((KG, C), jnp.float32),
                       pltpu.VMEM((KG,), jnp.int32),
                       pltpu.VMEM((KG,), jnp.int32),
                       pltpu.VMEM((64, C), jnp.float32),
                       pltpu.VMEM_SHARED((accmax, C), jnp.float32),
                       pltpu.SemaphoreType.DMA])
    def scatter_kernel(*refs):
        in_refs = refs[:nin]
        p_refs = refs[nin:nin + nr]
        mrows, ridx, sidx, zeros_v, acc, sem = refs[nin + nr:]
        cid = lax.axis_index("c")
        sid = lax.axis_index("s")
        wid = sid * NC + cid

        @pl.loop(0, 64)
        def _(r):
            @pl.loop(0, C, step=16)
            def _(cc):
                zeros_v[r, pl.ds(cc, 16)] = jnp.zeros((16,), jnp.float32)

        k = 0
        for r in range(nr):
            jobs = rank_jobs[r]
            job_refs = [(in_refs[k + 2 * t], in_refs[k + 2 * t + 1])
                        for t in range(len(jobs))]
            k += 2 * len(jobs)
            ch, accr = chs[r], accs[r]
            nch = npads[r] // ch
            srows = accr // NS
            erows = ch // NS
            for c in range(nch):
                base_node = c * ch

                @pl.loop(0, srows, step=64)
                def _(r0, srows=srows):
                    pltpu.sync_copy(zeros_v, acc.at[pl.ds(sid * srows + r0, 64)])
                plsc.subcore_barrier()

                for t, (m_ref, r_ref) in enumerate(job_refs):
                    epad = int(rank_jobs[r][t][0].shape[0])
                    nblk = epad // (NW * KG)

                    @pl.loop(0, nblk)
                    def _(i, m_ref=m_ref, r_ref=r_ref, nblk=nblk,
                          base_node=base_node, ch=ch):
                        base = (wid * nblk + i) * KG
                        pltpu.sync_copy(m_ref.at[pl.ds(base, KG)], mrows)
                        pltpu.sync_copy(r_ref.at[pl.ds(base, KG)], ridx)

                        @pl.loop(0, KG, step=16)
                        def _(tt):
                            rv = ridx[pl.ds(tt, 16)] - base_node
                            ok = (rv >= 0) & (rv < ch)
                            sidx[pl.ds(tt, 16)] = jnp.where(ok, rv, ch)

                        pltpu.sync_copy(mrows, acc.at[sidx], add=True)
                plsc.subcore_barrier()

                pltpu.sync_copy(
                    acc.at[pl.ds(sid * erows, erows)],
                    p_refs[r].at[cid, pl.ds(base_node + sid * erows, erows)])
                plsc.subcore_barrier()

    return scatter_kernel(*flat_in)


def _pad1(a, n, val):
    e = a.shape[0]
    if e == n:
        return a
    return jnp.concatenate(
        [a, jnp.full((n - e,) + a.shape[1:], val, a.dtype)], axis=0)


def kernel(x_0, x_1, x_2, adj_0, adj_1, inc_1, inc_2,
           inv_rr_0, inv_rr_1, inv_rrm1_1, inv_rrm1_2, params):
    p = params
    zb = jnp.zeros((C,), jnp.float32)

    a1 = _mm(x_0, p['same_0']['W1'][:C], zb, 1000)
    b1t = _mm(x_0, p['same_0']['W1'][C:2 * C], p['same_0']['b1'], 1000)
    a3 = _mm(x_0, p['l2h_1']['W1'][:C], zb, 1000)
    a2 = _mm(x_1, p['same_1']['W1'][:C], zb, 1000)
    b2t = _mm(x_1, p['same_1']['W1'][C:2 * C], p['same_1']['b1'], 1000)
    b3 = _mm(x_1, p['l2h_1']['W1'][C:2 * C], p['l2h_1']['b1'], 1000)
    a4 = _mm(x_1, p['l2h_2']['W1'][:C], zb, 1000)
    b4 = _mm(x_2, p['l2h_2']['W1'][C:2 * C], p['l2h_2']['b1'], 1000)

    convs = [
        dict(send=adj_0[0], recv=adj_0[1], inv=inv_rr_0, kinv=3,
             ta=a1, tb=b1t, p=p['same_0'], rank=0),
        dict(send=adj_1[0], recv=adj_1[1], inv=inv_rr_1, kinv=6,
             ta=a2, tb=b2t, p=p['same_1'], rank=1),
        dict(send=inc_1[0], recv=inc_1[1], inv=inv_rrm1_1, kinv=3,
             ta=a3, tb=b3, p=p['l2h_1'], rank=1),
        dict(send=inc_2[0], recv=inc_2[1], inv=inv_rrm1_2, kinv=6,
             ta=a4, tb=b4, p=p['l2h_2'], rank=2),
    ]

    gt, gi = [], []
    for cv in convs:
        e = cv['send'].shape[0]
        ep = _ceil_to(e, NW * KG)
        cv['epad'] = ep
        cv['send_p'] = _pad1(cv['send'], ep, 0)
        cv['recv_g'] = _pad1(cv['recv'], ep, 0)
        cv['recv_s'] = _pad1(cv['recv'], ep, BIG)
        invT = jnp.transpose(cv['inv'])
        invT = jnp.concatenate(
            [invT, jnp.zeros((8 - cv['kinv'], e), jnp.float32)], axis=0)
        cv['invT'] = _pad1(invT.T, ep, 0.0).T
        gt.extend((cv['ta'], cv['tb']))
        gi.extend((cv['send_p'], cv['recv_g']))

    gout = _sc_gather(gt, gi)

    for j, cv in enumerate(convs):
        cv['m'] = _edge_stage(gout[2 * j], gout[2 * j + 1],
                              cv['invT'], cv['p'], cv['kinv'])

    sizes = [N0, N1, N2]
    chs, accs, npads, rank_jobs = [], [], [], []
    for r in range(3):
        jobs = [(cv['m'], cv['recv_s']) for cv in convs if cv['rank'] == r]
        nch = max(1, -(-sizes[r] * 4 * C // (5 << 20)))
        ch = _ceil_to(-(-sizes[r] // nch), NS * 8)
        acc = _ceil_to(ch + 1, 1024)
        chs.append(ch)
        accs.append(acc)
        npads.append(nch * ch)
        rank_jobs.append(jobs)
    parts = _sc_scatter(rank_jobs, npads, chs, accs)

    feats = [x_0, x_1, x_2]
    outs = []
    for r in range(3):
        u = p['upd_%d' % r]
        outs.append(_update(parts[r], feats[r], u, 1000))
    return tuple(outs)
